# Initial kernel scaffold; baseline (speedup 1.0000x reference)
import jax
import jax.numpy as jnp
from jax.experimental import pallas as pl


def kernel(geom, aux, batch, W1, b1, bn1_g, bn1_b, W2, b2, bn2_g, bn2_b, Wa1, ba1, Wa2, ba2, ln_g, ln_b):
    raise NotImplementedError("write your pallas kernel here")


# speedup vs baseline: 1.9333x; 1.9333x over previous
import functools

import jax
import jax.numpy as jnp
from jax import lax
from jax.experimental import pallas as pl
from jax.experimental.pallas import tpu as pltpu
from jax.experimental.pallas import tpu_sc as plsc

N = 10000
NP = 10240
K = 20
D = 128
AD = 16
T = 256
GRID = (NP + T - 1) // T
EP = K * NP
TW = 256
E_REAL = N * K


def _knn_tables_body(geomT_ref, g_ref, aux_ref, brow_ref, bcol_ref,
                     W1_ref, b1_ref, Wa1_ref, ba1_ref,
                     nbr_ref, tq_ref, tg_ref, a2_ref, a1_ref):
    pid = pl.program_id(0)
    g = g_ref[...]
    geomT = geomT_ref[...]
    sq_col = jnp.sum(geomT * geomT, axis=0, keepdims=True)
    sq_row = jnp.sum(g * g, axis=1, keepdims=True)
    d = sq_row + sq_col - 2.0 * jnp.dot(g, geomT, preferred_element_type=jnp.float32)
    col = lax.broadcasted_iota(jnp.int32, (1, N), 1)
    rowg = pid * T + lax.broadcasted_iota(jnp.int32, (T, 1), 0)
    bad = (brow_ref[...] != bcol_ref[...]) | (col == rowg)
    d = jnp.where(bad, jnp.inf, d)
    lane = lax.broadcasted_iota(jnp.int32, (1, 32), 1)
    BIG = jnp.int32(2 ** 30)

    def body(k, carry):
        dk, acc = carry
        mn = jnp.min(dk, axis=1, keepdims=True)
        cand = jnp.where(dk == mn, col, BIG)
        idx = jnp.min(cand, axis=1, keepdims=True)
        acc = jnp.where(lane == k, idx, acc)
        dk = jnp.where(col == idx, jnp.inf, dk)
        return dk, acc

    _, acc = lax.fori_loop(0, K, body, (d, jnp.zeros((T, 32), jnp.int32)))
    nbr_ref[...] = acc

    W1 = W1_ref[...]
    P = jnp.dot(g, W1[:D], preferred_element_type=jnp.float32)
    Q = jnp.dot(g, W1[D:], preferred_element_type=jnp.float32)
    tq_ref[...] = Q
    tg_ref[...] = P - Q + b1_ref[...]
    aux = aux_ref[...]
    Wa1 = Wa1_ref[...]
    a2_ref[...] = jnp.dot(aux, Wa1[AD:], preferred_element_type=jnp.float32)
    a1_ref[...] = jnp.dot(aux, Wa1[:AD], preferred_element_type=jnp.float32) + ba1_ref[...]


def _knn_tables(geomT, geom, aux, brow, bcol, W1, b1r, Wa1, ba1r, interpret=False):
    full = lambda shape: pl.BlockSpec(shape, lambda i: (0, 0))
    return pl.pallas_call(
        _knn_tables_body,
        grid=(GRID,),
        in_specs=[
            full((D, N)),
            pl.BlockSpec((T, D), lambda i: (i, 0)),
            pl.BlockSpec((T, AD), lambda i: (i, 0)),
            pl.BlockSpec((T, 1), lambda i: (i, 0)),
            full((1, N)),
            full((2 * D, D)),
            full((1, D)),
            full((2 * AD, 64)),
            full((1, 64)),
        ],
        out_specs=[
            pl.BlockSpec((T, 32), lambda i: (i, 0)),
            pl.BlockSpec((T, D), lambda i: (i, 0)),
            pl.BlockSpec((T, D), lambda i: (i, 0)),
            pl.BlockSpec((T, 64), lambda i: (i, 0)),
            pl.BlockSpec((T, 64), lambda i: (i, 0)),
        ],
        out_shape=[
            jax.ShapeDtypeStruct((N, 32), jnp.int32),
            jax.ShapeDtypeStruct((N, D), jnp.float32),
            jax.ShapeDtypeStruct((N, D), jnp.float32),
            jax.ShapeDtypeStruct((N, 64), jnp.float32),
            jax.ShapeDtypeStruct((N, 64), jnp.float32),
        ],
        interpret=interpret,
    )(geomT, geom, aux, brow, bcol, W1, b1r, Wa1, ba1r)


_CHUNK = 320


def _sc_gather(table, idx_all):
    info = plsc.get_sparse_core_info()
    NC, NS = info.num_cores, info.num_subcores
    NW = NC * NS
    per_w = EP // NW
    n_chunks = per_w // _CHUNK
    mesh = plsc.VectorSubcoreMesh(core_axis_name="c", subcore_axis_name="s")

    @functools.partial(
        pl.kernel, mesh=mesh,
        out_type=(jax.ShapeDtypeStruct((EP, TW), jnp.float32),
                  jax.ShapeDtypeStruct((EP, TW), jnp.float32)),
        scratch_types=[
            pltpu.VMEM((_CHUNK,), jnp.int32),
            pltpu.VMEM((_CHUNK, TW), jnp.float32),
            pltpu.SemaphoreType.DMA,
        ],
    )
    def k(table_hbm, idx_hbm, outs_hbm, outt_hbm, idx_v, rows_v, sem):
        wid = lax.axis_index("s") * NC + lax.axis_index("c")
        base0 = wid * per_w

        @pl.loop(0, n_chunks)
        def _(c):
            b = base0 + c * _CHUNK
            pltpu.sync_copy(idx_hbm.at[pl.ds(b, _CHUNK)], idx_v)
            pltpu.async_copy(table_hbm.at[idx_v], rows_v, sem).wait()
            pltpu.sync_copy(rows_v, outs_hbm.at[pl.ds(b, _CHUNK)])

        @pl.loop(0, n_chunks)
        def _(c):
            b = base0 + c * _CHUNK
            pltpu.sync_copy(idx_hbm.at[pl.ds(EP + b, _CHUNK)], idx_v)
            pltpu.async_copy(table_hbm.at[idx_v], rows_v, sem).wait()
            pltpu.sync_copy(rows_v, outt_hbm.at[pl.ds(b, _CHUNK)])

    return k(table, idx_all)


def _stats1_body(gs_ref, gt_ref, s_ref, q_ref):
    pid = pl.program_id(0)
    h = gs_ref[...] + gt_ref[...]
    nid = pid * T + lax.broadcasted_iota(jnp.int32, (1, T, 1), 1)
    h = jnp.where(nid < N, h, 0.0)
    s_ref[...] = jnp.sum(h, axis=(0, 1))[None, None, :]
    q_ref[...] = jnp.sum(h * h, axis=(0, 1))[None, None, :]


def _stats1(Gs3, Gt3, interpret=False):
    return pl.pallas_call(
        _stats1_body,
        grid=(GRID,),
        in_specs=[
            pl.BlockSpec((K, T, D), lambda i: (0, i, 0)),
            pl.BlockSpec((K, T, D), lambda i: (0, i, 0)),
        ],
        out_specs=[
            pl.BlockSpec((1, 1, D), lambda i: (i, 0, 0)),
            pl.BlockSpec((1, 1, D), lambda i: (i, 0, 0)),
        ],
        out_shape=[
            jax.ShapeDtypeStruct((GRID, 1, D), jnp.float32),
            jax.ShapeDtypeStruct((GRID, 1, D), jnp.float32),
        ],
        interpret=interpret,
    )(Gs3, Gt3)


def _stats2_body(gs_ref, gt_ref, sc1_ref, sh1_ref, W2_ref, b2_ref, s_ref, q_ref):
    pid = pl.program_id(0)
    gs = gs_ref[...]
    gt = gt_ref[...]
    h1 = (gs[:, :, :D] + gt[:, :, :D]).reshape(K * T, D)
    e1 = jnp.maximum(h1 * sc1_ref[...] + sh1_ref[...], 0.0)
    h2 = jnp.dot(e1, W2_ref[...], preferred_element_type=jnp.float32) + b2_ref[...]
    nloc = lax.broadcasted_iota(jnp.int32, (K * T, 1), 0) % T
    h2 = jnp.where(pid * T + nloc < N, h2, 0.0)
    s_ref[...] = jnp.sum(h2, axis=0)[None, None, :]
    q_ref[...] = jnp.sum(h2 * h2, axis=0)[None, None, :]


def _stats2(Gs3, Gt3, sc1, sh1, W2, b2r, interpret=False):
    full2 = lambda shape: pl.BlockSpec(shape, lambda i: (0, 0))
    return pl.pallas_call(
        _stats2_body,
        grid=(GRID,),
        in_specs=[
            pl.BlockSpec((K, T, TW), lambda i: (0, i, 0)),
            pl.BlockSpec((K, T, TW), lambda i: (0, i, 0)),
            full2((1, D)), full2((1, D)), full2((D, D)), full2((1, D)),
        ],
        out_specs=[
            pl.BlockSpec((1, 1, D), lambda i: (i, 0, 0)),
            pl.BlockSpec((1, 1, D), lambda i: (i, 0, 0)),
        ],
        out_shape=[
            jax.ShapeDtypeStruct((GRID, 1, D), jnp.float32),
            jax.ShapeDtypeStruct((GRID, 1, D), jnp.float32),
        ],
        interpret=interpret,
    )(Gs3, Gt3, sc1, sh1, W2, b2r)


def _final_body(gs_ref, gt_ref, sc1_ref, sh1_ref, sc2_ref, sh2_ref,
                W2_ref, b2_ref, Wa2_ref, ba2_ref, lng_ref, lnb_ref, out_ref):
    gs = gs_ref[...]
    gt = gt_ref[...]
    h1 = (gs[:, :, :D] + gt[:, :, :D]).reshape(K * T, D)
    e1 = jnp.maximum(h1 * sc1_ref[...] + sh1_ref[...], 0.0)
    h2 = jnp.dot(e1, W2_ref[...], preferred_element_type=jnp.float32) + b2_ref[...]
    ef = jnp.maximum(h2 * sc2_ref[...] + sh2_ref[...], 0.0)
    a = jnp.maximum((gs[:, :, D:D + 64] + gt[:, :, D:D + 64]).reshape(K * T, 64), 0.0)
    gb = jnp.dot(a, Wa2_ref[...], preferred_element_type=jnp.float32) + ba2_ref[...]
    gamma = jax.nn.sigmoid(gb[:, :D] + 1.0)
    beta = gb[:, D:]
    mod = gamma * ef + beta
    mx = jnp.max(mod.reshape(K, T, D), axis=0)
    mu = jnp.mean(mx, axis=1, keepdims=True)
    xc = mx - mu
    var = jnp.mean(xc * xc, axis=1, keepdims=True)
    y = xc / jnp.sqrt(var + 1e-5) * lng_ref[...] + lnb_ref[...]
    out_ref[...] = jnp.maximum(y, 0.0)


def _final(Gs3, Gt3, sc1, sh1, sc2, sh2, W2, b2r, Wa2, ba2r, lngr, lnbr,
           interpret=False):
    full2 = lambda shape: pl.BlockSpec(shape, lambda i: (0, 0))
    return pl.pallas_call(
        _final_body,
        grid=(GRID,),
        in_specs=[
            pl.BlockSpec((K, T, TW), lambda i: (0, i, 0)),
            pl.BlockSpec((K, T, TW), lambda i: (0, i, 0)),
            full2((1, D)), full2((1, D)), full2((1, D)), full2((1, D)),
            full2((D, D)), full2((1, D)),
            full2((64, 2 * D)), full2((1, 2 * D)),
            full2((1, D)), full2((1, D)),
        ],
        out_specs=pl.BlockSpec((T, D), lambda i: (i, 0)),
        out_shape=jax.ShapeDtypeStruct((N, D), jnp.float32),
        interpret=interpret,
    )(Gs3, Gt3, sc1, sh1, sc2, sh2, W2, b2r, Wa2, ba2r, lngr, lnbr)


def kernel(geom, aux, batch, W1, b1, bn1_g, bn1_b, W2, b2, bn2_g, bn2_b,
           Wa1, ba1, Wa2, ba2, ln_g, ln_b):
    f32 = jnp.float32
    batch_i = batch.astype(jnp.int32)
    nbr, TQ, TG, A2, A1 = _knn_tables(
        geom.T, geom, aux, batch_i.reshape(N, 1), batch_i.reshape(1, N),
        W1, b1.reshape(1, D), Wa1, ba1.reshape(1, 64))

    zpad = jnp.zeros((N, TW - D - 64), jnp.float32)
    tab = jnp.concatenate([
        jnp.concatenate([TQ, A2, zpad], axis=1),
        jnp.concatenate([TG, A1, zpad], axis=1),
    ], axis=0)

    nbr_p = jnp.pad(nbr[:, :K], ((0, NP - N), (0, 0)))
    src_idx = jnp.clip(nbr_p.T.reshape(-1), 0, N - 1)
    tgt_idx = jnp.minimum(
        jnp.broadcast_to(jnp.arange(NP, dtype=jnp.int32)[None, :], (K, NP)),
        N - 1).reshape(-1) + N
    idx_all = jnp.concatenate([src_idx, tgt_idx]).astype(jnp.int32)

    Gs, Gt = _sc_gather(tab, idx_all)
    Gs3 = Gs.reshape(K, NP, TW)
    Gt3 = Gt.reshape(K, NP, TW)

    s1, q1 = _stats1(Gs3, Gt3)
    S1 = jnp.sum(s1, axis=0)[0]
    Q1 = jnp.sum(q1, axis=0)[0]
    mu1 = S1 / E_REAL
    var1 = Q1 / E_REAL - mu1 * mu1
    sc1 = bn1_g / jnp.sqrt(var1 + 1e-5)
    sh1 = bn1_b - mu1 * sc1

    s2, q2 = _stats2(Gs3, Gt3, sc1.reshape(1, D), sh1.reshape(1, D),
                     W2, b2.reshape(1, D))
    S2 = jnp.sum(s2, axis=0)[0]
    Q2 = jnp.sum(q2, axis=0)[0]
    mu2 = S2 / E_REAL
    var2 = Q2 / E_REAL - mu2 * mu2
    sc2 = bn2_g / jnp.sqrt(var2 + 1e-5)
    sh2 = bn2_b - mu2 * sc2

    return _final(Gs3, Gt3, sc1.reshape(1, D), sh1.reshape(1, D),
                  sc2.reshape(1, D), sh2.reshape(1, D), W2, b2.reshape(1, D),
                  Wa2, ba2.reshape(1, 2 * D), ln_g.reshape(1, D),
                  ln_b.reshape(1, D))


# speedup vs baseline: 2.1619x; 1.1183x over previous
import functools

import jax
import jax.numpy as jnp
from jax import lax
from jax.experimental import pallas as pl
from jax.experimental.pallas import tpu as pltpu
from jax.experimental.pallas import tpu_sc as plsc

N = 10000
NP = 10240
K = 20
D = 128
AD = 16
T = 256
GRID = (NP + T - 1) // T
EP = K * NP
TW = 256
E_REAL = N * K


def _knn_tables_body(geomT_ref, g_ref, aux_ref, brow_ref, bcol_ref,
                     W1_ref, b1_ref, Wa1_ref, ba1_ref,
                     nbr_ref, tq_ref, tg_ref, a2_ref, a1_ref):
    pid = pl.program_id(0)
    g = g_ref[...]
    geomT = geomT_ref[...]
    sq_col = jnp.sum(geomT * geomT, axis=0, keepdims=True)
    sq_row = jnp.sum(g * g, axis=1, keepdims=True)
    d = sq_row + sq_col - 2.0 * jnp.dot(g, geomT, preferred_element_type=jnp.float32)
    col = lax.broadcasted_iota(jnp.int32, (1, N), 1)
    rowg = pid * T + lax.broadcasted_iota(jnp.int32, (T, 1), 0)
    bad = (brow_ref[...] != bcol_ref[...]) | (col == rowg)
    d = jnp.where(bad, jnp.inf, d)
    lane = lax.broadcasted_iota(jnp.int32, (1, 32), 1)
    BIG = jnp.int32(2 ** 30)

    def body(k, carry):
        dk, acc = carry
        mn = jnp.min(dk, axis=1, keepdims=True)
        cand = jnp.where(dk == mn, col, BIG)
        idx = jnp.min(cand, axis=1, keepdims=True)
        acc = jnp.where(lane == k, idx, acc)
        dk = jnp.where(col == idx, jnp.inf, dk)
        return dk, acc

    _, acc = lax.fori_loop(0, K, body, (d, jnp.zeros((T, 32), jnp.int32)))
    nbr_ref[...] = acc

    W1 = W1_ref[...]
    P = jnp.dot(g, W1[:D], preferred_element_type=jnp.float32)
    Q = jnp.dot(g, W1[D:], preferred_element_type=jnp.float32)
    tq_ref[...] = Q
    tg_ref[...] = P - Q + b1_ref[...]
    aux = aux_ref[...]
    Wa1 = Wa1_ref[...]
    a2_ref[...] = jnp.dot(aux, Wa1[AD:], preferred_element_type=jnp.float32)
    a1_ref[...] = jnp.dot(aux, Wa1[:AD], preferred_element_type=jnp.float32) + ba1_ref[...]


def _knn_tables(geomT, geom, aux, brow, bcol, W1, b1r, Wa1, ba1r, interpret=False):
    full = lambda shape: pl.BlockSpec(shape, lambda i: (0, 0))
    return pl.pallas_call(
        _knn_tables_body,
        grid=(GRID,),
        in_specs=[
            full((D, N)),
            pl.BlockSpec((T, D), lambda i: (i, 0)),
            pl.BlockSpec((T, AD), lambda i: (i, 0)),
            pl.BlockSpec((T, 1), lambda i: (i, 0)),
            full((1, N)),
            full((2 * D, D)),
            full((1, D)),
            full((2 * AD, 64)),
            full((1, 64)),
        ],
        out_specs=[
            pl.BlockSpec((T, 32), lambda i: (i, 0)),
            pl.BlockSpec((T, D), lambda i: (i, 0)),
            pl.BlockSpec((T, D), lambda i: (i, 0)),
            pl.BlockSpec((T, 64), lambda i: (i, 0)),
            pl.BlockSpec((T, 64), lambda i: (i, 0)),
        ],
        out_shape=[
            jax.ShapeDtypeStruct((N, 32), jnp.int32),
            jax.ShapeDtypeStruct((N, D), jnp.float32),
            jax.ShapeDtypeStruct((N, D), jnp.float32),
            jax.ShapeDtypeStruct((N, 64), jnp.float32),
            jax.ShapeDtypeStruct((N, 64), jnp.float32),
        ],
        interpret=interpret,
    )(geomT, geom, aux, brow, bcol, W1, b1r, Wa1, ba1r)


_CHUNK = 320


def _sc_gather(table, idx_all):
    info = plsc.get_sparse_core_info()
    NC, NS = info.num_cores, info.num_subcores
    NW = NC * NS
    per_w = EP // NW
    n_chunks = per_w // _CHUNK
    mesh = plsc.VectorSubcoreMesh(core_axis_name="c", subcore_axis_name="s")

    @functools.partial(
        pl.kernel, mesh=mesh,
        out_type=jax.ShapeDtypeStruct((EP, TW), jnp.float32),
        scratch_types=[
            pltpu.VMEM((_CHUNK,), jnp.int32),
            pltpu.VMEM((_CHUNK, TW), jnp.float32),
            pltpu.SemaphoreType.DMA,
        ],
    )
    def k(table_hbm, idx_hbm, outs_hbm, idx_v, rows_v, sem):
        wid = lax.axis_index("s") * NC + lax.axis_index("c")
        base0 = wid * per_w

        @pl.loop(0, n_chunks)
        def _(c):
            b = base0 + c * _CHUNK
            pltpu.sync_copy(idx_hbm.at[pl.ds(b, _CHUNK)], idx_v)
            pltpu.async_copy(table_hbm.at[idx_v], rows_v, sem).wait()
            pltpu.sync_copy(rows_v, outs_hbm.at[pl.ds(b, _CHUNK)])

    return k(table, idx_all)


def _stats1_body(gs_ref, gt_ref, s_ref, q_ref):
    pid = pl.program_id(0)
    h = gs_ref[...] + gt_ref[...][None, :, :]
    nid = pid * T + lax.broadcasted_iota(jnp.int32, (1, T, 1), 1)
    h = jnp.where(nid < N, h, 0.0)
    s_ref[...] = jnp.sum(h, axis=(0, 1))[None, None, :]
    q_ref[...] = jnp.sum(h * h, axis=(0, 1))[None, None, :]


def _stats1(Gs3, Gt3, interpret=False):
    return pl.pallas_call(
        _stats1_body,
        grid=(GRID,),
        in_specs=[
            pl.BlockSpec((K, T, D), lambda i: (0, i, 0)),
            pl.BlockSpec((T, D), lambda i: (i, 0)),
        ],
        out_specs=[
            pl.BlockSpec((1, 1, D), lambda i: (i, 0, 0)),
            pl.BlockSpec((1, 1, D), lambda i: (i, 0, 0)),
        ],
        out_shape=[
            jax.ShapeDtypeStruct((GRID, 1, D), jnp.float32),
            jax.ShapeDtypeStruct((GRID, 1, D), jnp.float32),
        ],
        interpret=interpret,
    )(Gs3, Gt3)


def _stats2_body(gs_ref, gt_ref, sc1_ref, sh1_ref, W2_ref, b2_ref, s_ref, q_ref):
    pid = pl.program_id(0)
    gs = gs_ref[...]
    gt = gt_ref[...]
    h1 = (gs[:, :, :D] + gt[None, :, :D]).reshape(K * T, D)
    e1 = jnp.maximum(h1 * sc1_ref[...] + sh1_ref[...], 0.0)
    h2 = jnp.dot(e1, W2_ref[...], preferred_element_type=jnp.float32) + b2_ref[...]
    nloc = lax.broadcasted_iota(jnp.int32, (K * T, 1), 0) % T
    h2 = jnp.where(pid * T + nloc < N, h2, 0.0)
    s_ref[...] = jnp.sum(h2, axis=0)[None, None, :]
    q_ref[...] = jnp.sum(h2 * h2, axis=0)[None, None, :]


def _stats2(Gs3, Gt3, sc1, sh1, W2, b2r, interpret=False):
    full2 = lambda shape: pl.BlockSpec(shape, lambda i: (0, 0))
    return pl.pallas_call(
        _stats2_body,
        grid=(GRID,),
        in_specs=[
            pl.BlockSpec((K, T, TW), lambda i: (0, i, 0)),
            pl.BlockSpec((T, TW), lambda i: (i, 0)),
            full2((1, D)), full2((1, D)), full2((D, D)), full2((1, D)),
        ],
        out_specs=[
            pl.BlockSpec((1, 1, D), lambda i: (i, 0, 0)),
            pl.BlockSpec((1, 1, D), lambda i: (i, 0, 0)),
        ],
        out_shape=[
            jax.ShapeDtypeStruct((GRID, 1, D), jnp.float32),
            jax.ShapeDtypeStruct((GRID, 1, D), jnp.float32),
        ],
        interpret=interpret,
    )(Gs3, Gt3, sc1, sh1, W2, b2r)


def _final_body(gs_ref, gt_ref, sc1_ref, sh1_ref, sc2_ref, sh2_ref,
                W2_ref, b2_ref, Wa2_ref, ba2_ref, lng_ref, lnb_ref, out_ref):
    gs = gs_ref[...]
    gt = gt_ref[...]
    h1 = (gs[:, :, :D] + gt[None, :, :D]).reshape(K * T, D)
    e1 = jnp.maximum(h1 * sc1_ref[...] + sh1_ref[...], 0.0)
    h2 = jnp.dot(e1, W2_ref[...], preferred_element_type=jnp.float32) + b2_ref[...]
    ef = jnp.maximum(h2 * sc2_ref[...] + sh2_ref[...], 0.0)
    a = jnp.maximum((gs[:, :, D:D + 64] + gt[None, :, D:D + 64]).reshape(K * T, 64), 0.0)
    gb = jnp.dot(a, Wa2_ref[...], preferred_element_type=jnp.float32) + ba2_ref[...]
    gamma = jax.nn.sigmoid(gb[:, :D] + 1.0)
    beta = gb[:, D:]
    mod = gamma * ef + beta
    mx = jnp.max(mod.reshape(K, T, D), axis=0)
    mu = jnp.mean(mx, axis=1, keepdims=True)
    xc = mx - mu
    var = jnp.mean(xc * xc, axis=1, keepdims=True)
    y = xc / jnp.sqrt(var + 1e-5) * lng_ref[...] + lnb_ref[...]
    out_ref[...] = jnp.maximum(y, 0.0)


def _final(Gs3, Gt3, sc1, sh1, sc2, sh2, W2, b2r, Wa2, ba2r, lngr, lnbr,
           interpret=False):
    full2 = lambda shape: pl.BlockSpec(shape, lambda i: (0, 0))
    return pl.pallas_call(
        _final_body,
        grid=(GRID,),
        in_specs=[
            pl.BlockSpec((K, T, TW), lambda i: (0, i, 0)),
            pl.BlockSpec((T, TW), lambda i: (i, 0)),
            full2((1, D)), full2((1, D)), full2((1, D)), full2((1, D)),
            full2((D, D)), full2((1, D)),
            full2((64, 2 * D)), full2((1, 2 * D)),
            full2((1, D)), full2((1, D)),
        ],
        out_specs=pl.BlockSpec((T, D), lambda i: (i, 0)),
        out_shape=jax.ShapeDtypeStruct((N, D), jnp.float32),
        interpret=interpret,
    )(Gs3, Gt3, sc1, sh1, sc2, sh2, W2, b2r, Wa2, ba2r, lngr, lnbr)


def kernel(geom, aux, batch, W1, b1, bn1_g, bn1_b, W2, b2, bn2_g, bn2_b,
           Wa1, ba1, Wa2, ba2, ln_g, ln_b):
    f32 = jnp.float32
    batch_i = batch.astype(jnp.int32)
    nbr, TQ, TG, A2, A1 = _knn_tables(
        geom.T, geom, aux, batch_i.reshape(N, 1), batch_i.reshape(1, N),
        W1, b1.reshape(1, D), Wa1, ba1.reshape(1, 64))

    zpad = jnp.zeros((N, TW - D - 64), jnp.float32)
    tab = jnp.concatenate([TQ, A2, zpad], axis=1)
    ttab = jnp.pad(jnp.concatenate([TG, A1, zpad], axis=1),
                   ((0, NP - N), (0, 0)))

    nbr_p = jnp.pad(nbr[:, :K], ((0, NP - N), (0, 0)))
    src_idx = jnp.clip(nbr_p.T.reshape(-1), 0, N - 1).astype(jnp.int32)

    Gs = _sc_gather(tab, src_idx)
    Gs3 = Gs.reshape(K, NP, TW)
    Gt3 = ttab

    s1, q1 = _stats1(Gs3, Gt3)
    S1 = jnp.sum(s1, axis=0)[0]
    Q1 = jnp.sum(q1, axis=0)[0]
    mu1 = S1 / E_REAL
    var1 = Q1 / E_REAL - mu1 * mu1
    sc1 = bn1_g / jnp.sqrt(var1 + 1e-5)
    sh1 = bn1_b - mu1 * sc1

    s2, q2 = _stats2(Gs3, Gt3, sc1.reshape(1, D), sh1.reshape(1, D),
                     W2, b2.reshape(1, D))
    S2 = jnp.sum(s2, axis=0)[0]
    Q2 = jnp.sum(q2, axis=0)[0]
    mu2 = S2 / E_REAL
    var2 = Q2 / E_REAL - mu2 * mu2
    sc2 = bn2_g / jnp.sqrt(var2 + 1e-5)
    sh2 = bn2_b - mu2 * sc2

    return _final(Gs3, Gt3, sc1.reshape(1, D), sh1.reshape(1, D),
                  sc2.reshape(1, D), sh2.reshape(1, D), W2, b2.reshape(1, D),
                  Wa2, ba2.reshape(1, 2 * D), ln_g.reshape(1, D),
                  ln_b.reshape(1, D))


# speedup vs baseline: 2.1629x; 1.0004x over previous
import functools

import jax
import jax.numpy as jnp
from jax import lax
from jax.experimental import pallas as pl
from jax.experimental.pallas import tpu as pltpu
from jax.experimental.pallas import tpu_sc as plsc

N = 10000
NP = 10240
K = 20
D = 128
AD = 16
T = 256
GRID = (NP + T - 1) // T
EP = K * NP
TW = 256
E_REAL = N * K


def _knn_tables_body(geomT_ref, g_ref, aux_ref, brow_ref, bcol_ref,
                     W1_ref, b1_ref, Wa1_ref, ba1_ref,
                     nbr_ref, tq_ref, tg_ref, a2_ref, a1_ref):
    pid = pl.program_id(0)
    g = g_ref[...]
    geomT = geomT_ref[...]
    sq_col = jnp.sum(geomT * geomT, axis=0, keepdims=True)
    sq_row = jnp.sum(g * g, axis=1, keepdims=True)
    d = sq_row + sq_col - 2.0 * jnp.dot(g, geomT, preferred_element_type=jnp.float32)
    col = lax.broadcasted_iota(jnp.int32, (1, N), 1)
    rowg = pid * T + lax.broadcasted_iota(jnp.int32, (T, 1), 0)
    bad = (brow_ref[...] != bcol_ref[...]) | (col == rowg)
    d = jnp.where(bad, jnp.inf, d)
    lane = lax.broadcasted_iota(jnp.int32, (1, 32), 1)
    BIG = jnp.int32(2 ** 30)

    def body(k, carry):
        dk, acc = carry
        mn = jnp.min(dk, axis=1, keepdims=True)
        cand = jnp.where(dk == mn, col, BIG)
        idx = jnp.min(cand, axis=1, keepdims=True)
        acc = jnp.where(lane == k, idx, acc)
        dk = jnp.where(col == idx, jnp.inf, dk)
        return dk, acc

    _, acc = lax.fori_loop(0, K, body, (d, jnp.zeros((T, 32), jnp.int32)))
    nbr_ref[...] = acc

    W1 = W1_ref[...]
    P = jnp.dot(g, W1[:D], preferred_element_type=jnp.float32)
    Q = jnp.dot(g, W1[D:], preferred_element_type=jnp.float32)
    tq_ref[...] = Q
    tg_ref[...] = P - Q + b1_ref[...]
    aux = aux_ref[...]
    Wa1 = Wa1_ref[...]
    a2_ref[...] = jnp.dot(aux, Wa1[AD:], preferred_element_type=jnp.float32)
    a1_ref[...] = jnp.dot(aux, Wa1[:AD], preferred_element_type=jnp.float32) + ba1_ref[...]


def _knn_tables(geomT, geom, aux, brow, bcol, W1, b1r, Wa1, ba1r, interpret=False):
    full = lambda shape: pl.BlockSpec(shape, lambda i: (0, 0))
    return pl.pallas_call(
        _knn_tables_body,
        grid=(GRID,),
        in_specs=[
            full((D, N)),
            pl.BlockSpec((T, D), lambda i: (i, 0)),
            pl.BlockSpec((T, AD), lambda i: (i, 0)),
            pl.BlockSpec((T, 1), lambda i: (i, 0)),
            full((1, N)),
            full((2 * D, D)),
            full((1, D)),
            full((2 * AD, 64)),
            full((1, 64)),
        ],
        out_specs=[
            pl.BlockSpec((T, 32), lambda i: (i, 0)),
            pl.BlockSpec((T, D), lambda i: (i, 0)),
            pl.BlockSpec((T, D), lambda i: (i, 0)),
            pl.BlockSpec((T, 64), lambda i: (i, 0)),
            pl.BlockSpec((T, 64), lambda i: (i, 0)),
        ],
        out_shape=[
            jax.ShapeDtypeStruct((N, 32), jnp.int32),
            jax.ShapeDtypeStruct((N, D), jnp.float32),
            jax.ShapeDtypeStruct((N, D), jnp.float32),
            jax.ShapeDtypeStruct((N, 64), jnp.float32),
            jax.ShapeDtypeStruct((N, 64), jnp.float32),
        ],
        interpret=interpret,
        compiler_params=pltpu.CompilerParams(dimension_semantics=("parallel",)),
    )(geomT, geom, aux, brow, bcol, W1, b1r, Wa1, ba1r)


_CHUNK = 320


def _sc_gather(table, idx_all):
    info = plsc.get_sparse_core_info()
    NC, NS = info.num_cores, info.num_subcores
    NW = NC * NS
    per_w = EP // NW
    n_chunks = per_w // _CHUNK
    mesh = plsc.VectorSubcoreMesh(core_axis_name="c", subcore_axis_name="s")

    @functools.partial(
        pl.kernel, mesh=mesh,
        out_type=jax.ShapeDtypeStruct((EP, TW), jnp.float32),
        scratch_types=[
            pltpu.VMEM((_CHUNK,), jnp.int32),
            pltpu.VMEM((_CHUNK, TW), jnp.float32),
            pltpu.SemaphoreType.DMA,
        ],
    )
    def k(table_hbm, idx_hbm, outs_hbm, idx_v, rows_v, sem):
        wid = lax.axis_index("s") * NC + lax.axis_index("c")
        base0 = wid * per_w

        @pl.loop(0, n_chunks)
        def _(c):
            b = base0 + c * _CHUNK
            pltpu.sync_copy(idx_hbm.at[pl.ds(b, _CHUNK)], idx_v)
            pltpu.async_copy(table_hbm.at[idx_v], rows_v, sem).wait()
            pltpu.sync_copy(rows_v, outs_hbm.at[pl.ds(b, _CHUNK)])

    return k(table, idx_all)


def _stats1_body(gs_ref, gt_ref, s_ref, q_ref):
    pid = pl.program_id(0)
    h = gs_ref[...] + gt_ref[...][None, :, :]
    nid = pid * T + lax.broadcasted_iota(jnp.int32, (1, T, 1), 1)
    h = jnp.where(nid < N, h, 0.0)
    s_ref[...] = jnp.sum(h, axis=(0, 1))[None, None, :]
    q_ref[...] = jnp.sum(h * h, axis=(0, 1))[None, None, :]


def _stats1(Gs3, Gt3, interpret=False):
    return pl.pallas_call(
        _stats1_body,
        grid=(GRID,),
        in_specs=[
            pl.BlockSpec((K, T, D), lambda i: (0, i, 0)),
            pl.BlockSpec((T, D), lambda i: (i, 0)),
        ],
        out_specs=[
            pl.BlockSpec((1, 1, D), lambda i: (i, 0, 0)),
            pl.BlockSpec((1, 1, D), lambda i: (i, 0, 0)),
        ],
        out_shape=[
            jax.ShapeDtypeStruct((GRID, 1, D), jnp.float32),
            jax.ShapeDtypeStruct((GRID, 1, D), jnp.float32),
        ],
        interpret=interpret,
        compiler_params=pltpu.CompilerParams(dimension_semantics=("parallel",)),
    )(Gs3, Gt3)


def _stats2_body(gs_ref, gt_ref, sc1_ref, sh1_ref, W2_ref, b2_ref, s_ref, q_ref):
    pid = pl.program_id(0)
    gs = gs_ref[...]
    gt = gt_ref[...]
    h1 = (gs[:, :, :D] + gt[None, :, :D]).reshape(K * T, D)
    e1 = jnp.maximum(h1 * sc1_ref[...] + sh1_ref[...], 0.0)
    h2 = jnp.dot(e1, W2_ref[...], preferred_element_type=jnp.float32) + b2_ref[...]
    nloc = lax.broadcasted_iota(jnp.int32, (K * T, 1), 0) % T
    h2 = jnp.where(pid * T + nloc < N, h2, 0.0)
    s_ref[...] = jnp.sum(h2, axis=0)[None, None, :]
    q_ref[...] = jnp.sum(h2 * h2, axis=0)[None, None, :]


def _stats2(Gs3, Gt3, sc1, sh1, W2, b2r, interpret=False):
    full2 = lambda shape: pl.BlockSpec(shape, lambda i: (0, 0))
    return pl.pallas_call(
        _stats2_body,
        grid=(GRID,),
        in_specs=[
            pl.BlockSpec((K, T, TW), lambda i: (0, i, 0)),
            pl.BlockSpec((T, TW), lambda i: (i, 0)),
            full2((1, D)), full2((1, D)), full2((D, D)), full2((1, D)),
        ],
        out_specs=[
            pl.BlockSpec((1, 1, D), lambda i: (i, 0, 0)),
            pl.BlockSpec((1, 1, D), lambda i: (i, 0, 0)),
        ],
        out_shape=[
            jax.ShapeDtypeStruct((GRID, 1, D), jnp.float32),
            jax.ShapeDtypeStruct((GRID, 1, D), jnp.float32),
        ],
        interpret=interpret,
        compiler_params=pltpu.CompilerParams(dimension_semantics=("parallel",)),
    )(Gs3, Gt3, sc1, sh1, W2, b2r)


def _final_body(gs_ref, gt_ref, sc1_ref, sh1_ref, sc2_ref, sh2_ref,
                W2_ref, b2_ref, Wa2_ref, ba2_ref, lng_ref, lnb_ref, out_ref):
    gs = gs_ref[...]
    gt = gt_ref[...]
    h1 = (gs[:, :, :D] + gt[None, :, :D]).reshape(K * T, D)
    e1 = jnp.maximum(h1 * sc1_ref[...] + sh1_ref[...], 0.0)
    h2 = jnp.dot(e1, W2_ref[...], preferred_element_type=jnp.float32) + b2_ref[...]
    ef = jnp.maximum(h2 * sc2_ref[...] + sh2_ref[...], 0.0)
    a = jnp.maximum((gs[:, :, D:D + 64] + gt[None, :, D:D + 64]).reshape(K * T, 64), 0.0)
    gb = jnp.dot(a, Wa2_ref[...], preferred_element_type=jnp.float32) + ba2_ref[...]
    gamma = jax.nn.sigmoid(gb[:, :D] + 1.0)
    beta = gb[:, D:]
    mod = gamma * ef + beta
    mx = jnp.max(mod.reshape(K, T, D), axis=0)
    mu = jnp.mean(mx, axis=1, keepdims=True)
    xc = mx - mu
    var = jnp.mean(xc * xc, axis=1, keepdims=True)
    y = xc / jnp.sqrt(var + 1e-5) * lng_ref[...] + lnb_ref[...]
    out_ref[...] = jnp.maximum(y, 0.0)


def _final(Gs3, Gt3, sc1, sh1, sc2, sh2, W2, b2r, Wa2, ba2r, lngr, lnbr,
           interpret=False):
    full2 = lambda shape: pl.BlockSpec(shape, lambda i: (0, 0))
    return pl.pallas_call(
        _final_body,
        grid=(GRID,),
        in_specs=[
            pl.BlockSpec((K, T, TW), lambda i: (0, i, 0)),
            pl.BlockSpec((T, TW), lambda i: (i, 0)),
            full2((1, D)), full2((1, D)), full2((1, D)), full2((1, D)),
            full2((D, D)), full2((1, D)),
            full2((64, 2 * D)), full2((1, 2 * D)),
            full2((1, D)), full2((1, D)),
        ],
        out_specs=pl.BlockSpec((T, D), lambda i: (i, 0)),
        out_shape=jax.ShapeDtypeStruct((N, D), jnp.float32),
        interpret=interpret,
        compiler_params=pltpu.CompilerParams(dimension_semantics=("parallel",)),
    )(Gs3, Gt3, sc1, sh1, sc2, sh2, W2, b2r, Wa2, ba2r, lngr, lnbr)


def kernel(geom, aux, batch, W1, b1, bn1_g, bn1_b, W2, b2, bn2_g, bn2_b,
           Wa1, ba1, Wa2, ba2, ln_g, ln_b):
    f32 = jnp.float32
    batch_i = batch.astype(jnp.int32)
    nbr, TQ, TG, A2, A1 = _knn_tables(
        geom.T, geom, aux, batch_i.reshape(N, 1), batch_i.reshape(1, N),
        W1, b1.reshape(1, D), Wa1, ba1.reshape(1, 64))

    zpad = jnp.zeros((N, TW - D - 64), jnp.float32)
    tab = jnp.concatenate([TQ, A2, zpad], axis=1)
    ttab = jnp.pad(jnp.concatenate([TG, A1, zpad], axis=1),
                   ((0, NP - N), (0, 0)))

    nbr_p = jnp.pad(nbr[:, :K], ((0, NP - N), (0, 0)))
    src_idx = jnp.clip(nbr_p.T.reshape(-1), 0, N - 1).astype(jnp.int32)

    Gs = _sc_gather(tab, src_idx)
    Gs3 = Gs.reshape(K, NP, TW)
    Gt3 = ttab

    s1, q1 = _stats1(Gs3, Gt3)
    S1 = jnp.sum(s1, axis=0)[0]
    Q1 = jnp.sum(q1, axis=0)[0]
    mu1 = S1 / E_REAL
    var1 = Q1 / E_REAL - mu1 * mu1
    sc1 = bn1_g / jnp.sqrt(var1 + 1e-5)
    sh1 = bn1_b - mu1 * sc1

    s2, q2 = _stats2(Gs3, Gt3, sc1.reshape(1, D), sh1.reshape(1, D),
                     W2, b2.reshape(1, D))
    S2 = jnp.sum(s2, axis=0)[0]
    Q2 = jnp.sum(q2, axis=0)[0]
    mu2 = S2 / E_REAL
    var2 = Q2 / E_REAL - mu2 * mu2
    sc2 = bn2_g / jnp.sqrt(var2 + 1e-5)
    sh2 = bn2_b - mu2 * sc2

    return _final(Gs3, Gt3, sc1.reshape(1, D), sh1.reshape(1, D),
                  sc2.reshape(1, D), sh2.reshape(1, D), W2, b2.reshape(1, D),
                  Wa2, ba2.reshape(1, 2 * D), ln_g.reshape(1, D),
                  ln_b.reshape(1, D))


# speedup vs baseline: 4.3699x; 2.0204x over previous
import functools

import jax
import jax.numpy as jnp
from jax import lax
from jax.experimental import pallas as pl
from jax.experimental.pallas import tpu as pltpu
from jax.experimental.pallas import tpu_sc as plsc

N = 10000
NP = 10240
K = 20
D = 128
AD = 16
T = 256
GRID = (NP + T - 1) // T
EP = K * NP
TW = 256
E_REAL = N * K
WIN = 4096


def _knn_tables_body(win_ref, geomT_ref, g_ref, aux_ref, brow_ref, bcol_ref,
                     W1_ref, b1_ref, Wa1_ref, ba1_ref,
                     nbr_ref, tq_ref, tg_ref, a2_ref, a1_ref):
    pid = pl.program_id(0)
    g = g_ref[...]
    sq_row = jnp.sum(g * g, axis=1, keepdims=True)
    rowg = pid * T + lax.broadcasted_iota(jnp.int32, (T, 1), 0)
    brow = brow_ref[...]
    lane = lax.broadcasted_iota(jnp.int32, (1, 32), 1)
    BIG = jnp.int32(2 ** 30)

    def topk(d, colids):
        def body(k, carry):
            dk, acc = carry
            mn = jnp.min(dk, axis=1, keepdims=True)
            cand = jnp.where(dk == mn, colids, BIG)
            idx = jnp.min(cand, axis=1, keepdims=True)
            acc = jnp.where(lane == k, idx, acc)
            dk = jnp.where(colids == idx, jnp.inf, dk)
            return dk, acc
        _, acc = lax.fori_loop(0, K, body, (d, jnp.zeros((T, 32), jnp.int32)))
        return acc

    ws = pl.multiple_of(win_ref[pid, 0], 512)
    sel_full = win_ref[pid, 1] == 1

    @pl.when(sel_full)
    def _():
        geomT = geomT_ref[...]
        sq_col = jnp.sum(geomT * geomT, axis=0, keepdims=True)
        d = sq_row + sq_col - 2.0 * jnp.dot(g, geomT, preferred_element_type=jnp.float32)
        col = lax.broadcasted_iota(jnp.int32, (1, NP), 1)
        bad = (brow != bcol_ref[...]) | (col == rowg)
        nbr_ref[...] = topk(jnp.where(bad, jnp.inf, d), col)

    @pl.when(jnp.logical_not(sel_full))
    def _():
        gw = geomT_ref[:, pl.ds(ws, WIN)]
        sq_col = jnp.sum(gw * gw, axis=0, keepdims=True)
        d = sq_row + sq_col - 2.0 * jnp.dot(g, gw, preferred_element_type=jnp.float32)
        col = ws + lax.broadcasted_iota(jnp.int32, (1, WIN), 1)
        bcw = bcol_ref[:, pl.ds(ws, WIN)]
        bad = (brow != bcw) | (col == rowg)
        nbr_ref[...] = topk(jnp.where(bad, jnp.inf, d), col)

    W1 = W1_ref[...]
    P = jnp.dot(g, W1[:D], preferred_element_type=jnp.float32)
    Q = jnp.dot(g, W1[D:], preferred_element_type=jnp.float32)
    tq_ref[...] = Q
    tg_ref[...] = P - Q + b1_ref[...]
    aux = aux_ref[...]
    Wa1 = Wa1_ref[...]
    a2_ref[...] = jnp.dot(aux, Wa1[AD:], preferred_element_type=jnp.float32)
    a1_ref[...] = jnp.dot(aux, Wa1[:AD], preferred_element_type=jnp.float32) + ba1_ref[...]


def _knn_tables(win, geomT, geom, aux, brow, bcol, W1, b1r, Wa1, ba1r,
                interpret=False):
    full = lambda shape: pl.BlockSpec(shape, lambda i: (0, 0))
    return pl.pallas_call(
        _knn_tables_body,
        grid=(GRID,),
        in_specs=[
            pl.BlockSpec(memory_space=pltpu.SMEM),
            full((D, NP)),
            pl.BlockSpec((T, D), lambda i: (i, 0)),
            pl.BlockSpec((T, AD), lambda i: (i, 0)),
            pl.BlockSpec((T, 1), lambda i: (i, 0)),
            full((1, NP)),
            full((2 * D, D)),
            full((1, D)),
            full((2 * AD, 64)),
            full((1, 64)),
        ],
        out_specs=[
            pl.BlockSpec((T, 32), lambda i: (i, 0)),
            pl.BlockSpec((T, D), lambda i: (i, 0)),
            pl.BlockSpec((T, D), lambda i: (i, 0)),
            pl.BlockSpec((T, 64), lambda i: (i, 0)),
            pl.BlockSpec((T, 64), lambda i: (i, 0)),
        ],
        out_shape=[
            jax.ShapeDtypeStruct((N, 32), jnp.int32),
            jax.ShapeDtypeStruct((N, D), jnp.float32),
            jax.ShapeDtypeStruct((N, D), jnp.float32),
            jax.ShapeDtypeStruct((N, 64), jnp.float32),
            jax.ShapeDtypeStruct((N, 64), jnp.float32),
        ],
        interpret=interpret,
        compiler_params=pltpu.CompilerParams(dimension_semantics=("parallel",)),
    )(win, geomT, geom, aux, brow, bcol, W1, b1r, Wa1, ba1r)


_CHUNK = 320


def _sc_gather(table, idx_all):
    info = plsc.get_sparse_core_info()
    NC, NS = info.num_cores, info.num_subcores
    NW = NC * NS
    per_w = EP // NW
    n_chunks = per_w // _CHUNK
    mesh = plsc.VectorSubcoreMesh(core_axis_name="c", subcore_axis_name="s")

    @functools.partial(
        pl.kernel, mesh=mesh,
        out_type=jax.ShapeDtypeStruct((EP, TW), jnp.float32),
        scratch_types=[
            pltpu.VMEM((_CHUNK,), jnp.int32),
            pltpu.VMEM((_CHUNK, TW), jnp.float32),
            pltpu.SemaphoreType.DMA,
        ],
    )
    def k(table_hbm, idx_hbm, outs_hbm, idx_v, rows_v, sem):
        wid = lax.axis_index("s") * NC + lax.axis_index("c")
        base0 = wid * per_w

        @pl.loop(0, n_chunks)
        def _(c):
            b = base0 + c * _CHUNK
            pltpu.sync_copy(idx_hbm.at[pl.ds(b, _CHUNK)], idx_v)
            pltpu.async_copy(table_hbm.at[idx_v], rows_v, sem).wait()
            pltpu.sync_copy(rows_v, outs_hbm.at[pl.ds(b, _CHUNK)])

    return k(table, idx_all)


def _stats1_body(gs_ref, gt_ref, s_ref, q_ref):
    pid = pl.program_id(0)
    h = gs_ref[...] + gt_ref[...][None, :, :]
    nid = pid * T + lax.broadcasted_iota(jnp.int32, (1, T, 1), 1)
    h = jnp.where(nid < N, h, 0.0)
    s_ref[...] = jnp.sum(h, axis=(0, 1))[None, None, :]
    q_ref[...] = jnp.sum(h * h, axis=(0, 1))[None, None, :]


def _stats1(Gs3, Gt3, interpret=False):
    return pl.pallas_call(
        _stats1_body,
        grid=(GRID,),
        in_specs=[
            pl.BlockSpec((K, T, D), lambda i: (0, i, 0)),
            pl.BlockSpec((T, D), lambda i: (i, 0)),
        ],
        out_specs=[
            pl.BlockSpec((1, 1, D), lambda i: (i, 0, 0)),
            pl.BlockSpec((1, 1, D), lambda i: (i, 0, 0)),
        ],
        out_shape=[
            jax.ShapeDtypeStruct((GRID, 1, D), jnp.float32),
            jax.ShapeDtypeStruct((GRID, 1, D), jnp.float32),
        ],
        interpret=interpret,
        compiler_params=pltpu.CompilerParams(dimension_semantics=("parallel",)),
    )(Gs3, Gt3)


def _stats2_body(gs_ref, gt_ref, sc1_ref, sh1_ref, W2_ref, b2_ref, s_ref, q_ref):
    pid = pl.program_id(0)
    gs = gs_ref[...]
    gt = gt_ref[...]
    h1 = (gs[:, :, :D] + gt[None, :, :D]).reshape(K * T, D)
    e1 = jnp.maximum(h1 * sc1_ref[...] + sh1_ref[...], 0.0)
    h2 = jnp.dot(e1, W2_ref[...], preferred_element_type=jnp.float32) + b2_ref[...]
    nloc = lax.broadcasted_iota(jnp.int32, (K * T, 1), 0) % T
    h2 = jnp.where(pid * T + nloc < N, h2, 0.0)
    s_ref[...] = jnp.sum(h2, axis=0)[None, None, :]
    q_ref[...] = jnp.sum(h2 * h2, axis=0)[None, None, :]


def _stats2(Gs3, Gt3, sc1, sh1, W2, b2r, interpret=False):
    full2 = lambda shape: pl.BlockSpec(shape, lambda i: (0, 0))
    return pl.pallas_call(
        _stats2_body,
        grid=(GRID,),
        in_specs=[
            pl.BlockSpec((K, T, TW), lambda i: (0, i, 0)),
            pl.BlockSpec((T, TW), lambda i: (i, 0)),
            full2((1, D)), full2((1, D)), full2((D, D)), full2((1, D)),
        ],
        out_specs=[
            pl.BlockSpec((1, 1, D), lambda i: (i, 0, 0)),
            pl.BlockSpec((1, 1, D), lambda i: (i, 0, 0)),
        ],
        out_shape=[
            jax.ShapeDtypeStruct((GRID, 1, D), jnp.float32),
            jax.ShapeDtypeStruct((GRID, 1, D), jnp.float32),
        ],
        interpret=interpret,
        compiler_params=pltpu.CompilerParams(dimension_semantics=("parallel",)),
    )(Gs3, Gt3, sc1, sh1, W2, b2r)


def _final_body(gs_ref, gt_ref, sc1_ref, sh1_ref, sc2_ref, sh2_ref,
                W2_ref, b2_ref, Wa2_ref, ba2_ref, lng_ref, lnb_ref, out_ref):
    gs = gs_ref[...]
    gt = gt_ref[...]
    h1 = (gs[:, :, :D] + gt[None, :, :D]).reshape(K * T, D)
    e1 = jnp.maximum(h1 * sc1_ref[...] + sh1_ref[...], 0.0)
    h2 = jnp.dot(e1, W2_ref[...], preferred_element_type=jnp.float32) + b2_ref[...]
    ef = jnp.maximum(h2 * sc2_ref[...] + sh2_ref[...], 0.0)
    a = jnp.maximum((gs[:, :, D:D + 64] + gt[None, :, D:D + 64]).reshape(K * T, 64), 0.0)
    gb = jnp.dot(a, Wa2_ref[...], preferred_element_type=jnp.float32) + ba2_ref[...]
    gamma = jax.nn.sigmoid(gb[:, :D] + 1.0)
    beta = gb[:, D:]
    mod = gamma * ef + beta
    mx = jnp.max(mod.reshape(K, T, D), axis=0)
    mu = jnp.mean(mx, axis=1, keepdims=True)
    xc = mx - mu
    var = jnp.mean(xc * xc, axis=1, keepdims=True)
    y = xc / jnp.sqrt(var + 1e-5) * lng_ref[...] + lnb_ref[...]
    out_ref[...] = jnp.maximum(y, 0.0)


def _final(Gs3, Gt3, sc1, sh1, sc2, sh2, W2, b2r, Wa2, ba2r, lngr, lnbr,
           interpret=False):
    full2 = lambda shape: pl.BlockSpec(shape, lambda i: (0, 0))
    return pl.pallas_call(
        _final_body,
        grid=(GRID,),
        in_specs=[
            pl.BlockSpec((K, T, TW), lambda i: (0, i, 0)),
            pl.BlockSpec((T, TW), lambda i: (i, 0)),
            full2((1, D)), full2((1, D)), full2((1, D)), full2((1, D)),
            full2((D, D)), full2((1, D)),
            full2((64, 2 * D)), full2((1, 2 * D)),
            full2((1, D)), full2((1, D)),
        ],
        out_specs=pl.BlockSpec((T, D), lambda i: (i, 0)),
        out_shape=jax.ShapeDtypeStruct((N, D), jnp.float32),
        interpret=interpret,
        compiler_params=pltpu.CompilerParams(dimension_semantics=("parallel",)),
    )(Gs3, Gt3, sc1, sh1, sc2, sh2, W2, b2r, Wa2, ba2r, lngr, lnbr)


def kernel(geom, aux, batch, W1, b1, bn1_g, bn1_b, W2, b2, bn2_g, bn2_b,
           Wa1, ba1, Wa2, ba2, ln_g, ln_b):
    f32 = jnp.float32
    batch_i = batch.astype(jnp.int32)
    rs = jnp.arange(GRID, dtype=jnp.int32) * T
    re = jnp.minimum(rs + T - 1, N - 1)
    bmin = batch_i[jnp.minimum(rs, N - 1)]
    bmax = batch_i[re]
    lo = jnp.searchsorted(batch_i, bmin, side='left').astype(jnp.int32)
    hi = jnp.searchsorted(batch_i, bmax, side='right').astype(jnp.int32)
    ws = jnp.clip((lo // 512) * 512, 0, NP - WIN)
    sel = ((hi - ws) > WIN).astype(jnp.int32)
    win = jnp.stack([ws, sel], axis=1)
    geomTp = jnp.pad(geom.T, ((0, 0), (0, NP - N)))
    bcolp = jnp.pad(batch_i.reshape(1, N), ((0, 0), (0, NP - N)),
                    constant_values=-1)
    nbr, TQ, TG, A2, A1 = _knn_tables(
        win, geomTp, geom, aux, batch_i.reshape(N, 1), bcolp,
        W1, b1.reshape(1, D), Wa1, ba1.reshape(1, 64))

    zpad = jnp.zeros((N, TW - D - 64), jnp.float32)
    tab = jnp.concatenate([TQ, A2, zpad], axis=1)
    ttab = jnp.pad(jnp.concatenate([TG, A1, zpad], axis=1),
                   ((0, NP - N), (0, 0)))

    nbr_p = jnp.pad(nbr[:, :K], ((0, NP - N), (0, 0)))
    src_idx = jnp.clip(nbr_p.T.reshape(-1), 0, N - 1).astype(jnp.int32)

    Gs = _sc_gather(tab, src_idx)
    Gs3 = Gs.reshape(K, NP, TW)
    Gt3 = ttab

    s1, q1 = _stats1(Gs3, Gt3)
    S1 = jnp.sum(s1, axis=0)[0]
    Q1 = jnp.sum(q1, axis=0)[0]
    mu1 = S1 / E_REAL
    var1 = Q1 / E_REAL - mu1 * mu1
    sc1 = bn1_g / jnp.sqrt(var1 + 1e-5)
    sh1 = bn1_b - mu1 * sc1

    s2, q2 = _stats2(Gs3, Gt3, sc1.reshape(1, D), sh1.reshape(1, D),
                     W2, b2.reshape(1, D))
    S2 = jnp.sum(s2, axis=0)[0]
    Q2 = jnp.sum(q2, axis=0)[0]
    mu2 = S2 / E_REAL
    var2 = Q2 / E_REAL - mu2 * mu2
    sc2 = bn2_g / jnp.sqrt(var2 + 1e-5)
    sh2 = bn2_b - mu2 * sc2

    return _final(Gs3, Gt3, sc1.reshape(1, D), sh1.reshape(1, D),
                  sc2.reshape(1, D), sh2.reshape(1, D), W2, b2.reshape(1, D),
                  Wa2, ba2.reshape(1, 2 * D), ln_g.reshape(1, D),
                  ln_b.reshape(1, D))
